# baseline (speedup 1.0000x reference)
import functools

import jax
import jax.numpy as jnp
from jax import lax
from jax.experimental import pallas as pl
from jax.experimental.pallas import tpu as pltpu
from jax.experimental.pallas import tpu_sc as plsc

VOC_SIZ = 100000
EMB_DIM = 64
NUM_POS = 2048
B, T = 64, 2048

_NC, _NS, _L = 2, 16, 16
_NW = _NC * _NS
_DPW = EMB_DIM // _NW
_NB = 4


def _emb_body(idx_hbm, table_hbm, pos_hbm, emb_out, tok_out, pos_out,
              row_v, pos_v, idx_bufs, tok_bufs, emb_bufs,
              s_row, s_pos, s_idx, s_tok, s_emb):
    wid = lax.axis_index("s") * _NC + lax.axis_index("c")

    def compute_pair(b0, j0):
        j1 = j0 + 1

        @plsc.parallel_loop(0, T, step=_L, unroll=8)
        def _grp(t):
            sl = pl.ds(t, _L)
            pv = pos_v[sl]
            tvA = plsc.load_gather(row_v, [idx_bufs[j0][sl]])
            tvB = plsc.load_gather(row_v, [idx_bufs[j1][sl]])
            tok_bufs[j0][sl] = tvA
            emb_bufs[j0][sl] = tvA + pv
            tok_bufs[j1][sl] = tvB
            emb_bufs[j1][sl] = tvB + pv

    def issue_writes(b0, j0, d):
        for o in range(2):
            pltpu.async_copy(tok_bufs[j0 + o], tok_out.at[b0 + o, d],
                             s_tok[j0 + o])
            pltpu.async_copy(emb_bufs[j0 + o], emb_out.at[b0 + o, d],
                             s_emb[j0 + o])

    def wait_writes(b0, j0, d):
        for o in range(2):
            pltpu.make_async_copy(
                tok_bufs[j0 + o], tok_out.at[b0 + o, d],
                s_tok[j0 + o]).wait()
            pltpu.make_async_copy(
                emb_bufs[j0 + o], emb_out.at[b0 + o, d],
                s_emb[j0 + o]).wait()

    def wait_idx(b0, j0):
        for o in range(2):
            pltpu.make_async_copy(
                idx_hbm.at[b0 + o], idx_bufs[j0 + o], s_idx[j0 + o]).wait()

    def issue_idx(b0, j0):
        for o in range(2):
            pltpu.async_copy(
                idx_hbm.at[b0 + o], idx_bufs[j0 + o], s_idx[j0 + o])

    for u in range(_DPW):
        d = wid + u * _NW
        cp_row = pltpu.async_copy(table_hbm.at[d], row_v, s_row)
        cp_pos = pltpu.async_copy(pos_hbm.at[d], pos_v, s_pos)
        issue_idx(0, 0)
        issue_idx(2, 2)
        cp_pos.wait()
        pltpu.sync_copy(pos_v, pos_out.at[d])
        cp_row.wait()

        for s in range(2):
            b0, j0 = 2 * s, 2 * s
            wait_idx(b0, j0)
            if u > 0:
                wait_writes(B - _NB + b0, j0, d - _NW)
            compute_pair(b0, j0)
            issue_writes(b0, j0, d)
            issue_idx(b0 + _NB, j0)

        def blk(ss, c, d=d):
            for par in range(2):
                b0 = ss * _NB + 2 * par
                j0 = 2 * par
                wait_idx(b0, j0)
                wait_writes(b0 - _NB, j0, d)
                compute_pair(b0, j0)
                issue_writes(b0, j0, d)

                @pl.when(b0 + _NB < B)
                def _():
                    issue_idx(b0 + _NB, j0)
            return c
        lax.fori_loop(1, B // _NB, blk, 0)

    d_last = wid + (_DPW - 1) * _NW
    wait_writes(B - _NB, 0, d_last)
    wait_writes(B - 2, 2, d_last)


@jax.jit
def _emb_call(idx2d, table_t, pos_t):
    mesh = plsc.VectorSubcoreMesh(core_axis_name="c", subcore_axis_name="s")
    out_type = (
        jax.ShapeDtypeStruct((B, EMB_DIM, T), jnp.float32),
        jax.ShapeDtypeStruct((B, EMB_DIM, T), jnp.float32),
        jax.ShapeDtypeStruct((EMB_DIM, NUM_POS), jnp.float32),
    )
    scratch = [
        pltpu.VMEM((VOC_SIZ,), jnp.float32),
        pltpu.VMEM((T,), jnp.float32),
        [pltpu.VMEM((T,), jnp.int32)] * _NB,
        [pltpu.VMEM((T,), jnp.float32)] * _NB,
        [pltpu.VMEM((T,), jnp.float32)] * _NB,
        pltpu.SemaphoreType.DMA,
        pltpu.SemaphoreType.DMA,
        [pltpu.SemaphoreType.DMA] * _NB,
        [pltpu.SemaphoreType.DMA] * _NB,
        [pltpu.SemaphoreType.DMA] * _NB,
    ]
    fn = functools.partial(
        pl.kernel, mesh=mesh, out_type=out_type, scratch_types=scratch,
        compiler_params=pltpu.CompilerParams(needs_layout_passes=False),
    )(_emb_body)
    return fn(idx2d, table_t, pos_t)


def kernel(batInpTokSeq, token_table, pos_table):
    idx2d = batInpTokSeq.astype(jnp.int32)
    emb, tok, pos = _emb_call(idx2d, token_table.T, pos_table.T)
    return (
        emb.transpose(0, 2, 1),
        tok.transpose(0, 2, 1),
        pos.T,
    )


# speedup vs baseline: 1.0709x; 1.0709x over previous
import functools

import jax
import jax.numpy as jnp
from jax import lax
from jax.experimental import pallas as pl
from jax.experimental.pallas import tpu as pltpu
from jax.experimental.pallas import tpu_sc as plsc

VOC_SIZ = 100000
EMB_DIM = 64
NUM_POS = 2048
B, T = 64, 2048

_NC, _NS, _L = 2, 16, 16
_NW = _NC * _NS
_DPW = EMB_DIM // _NW
_NIB = 4
_NOB = 2


def _emb_body(idx_hbm, table_hbm, pos_hbm, emb_out, tok_out, pos_out,
              row_v, pos_v, idx_bufs, tok_bufs, emb_bufs,
              s_row, s_pos, s_idx, s_tok, s_emb):
    wid = lax.axis_index("s") * _NC + lax.axis_index("c")

    def compute(b, jj, d):
        ob = jj % _NOB

        @plsc.parallel_loop(0, T, step=_L, unroll=8)
        def _grp(t):
            sl = pl.ds(t, _L)
            iv = idx_bufs[jj][sl]
            tv = plsc.load_gather(row_v, [iv])
            tok_bufs[ob][sl] = tv
            emb_bufs[ob][sl] = tv + pos_v[sl]

        pltpu.async_copy(tok_bufs[ob], tok_out.at[b, d], s_tok[ob])
        pltpu.async_copy(emb_bufs[ob], emb_out.at[b, d], s_emb[ob])

    def wait_writes(b, jj, d):
        ob = jj % _NOB
        pltpu.make_async_copy(tok_bufs[ob], tok_out.at[b, d], s_tok[ob]).wait()
        pltpu.make_async_copy(emb_bufs[ob], emb_out.at[b, d], s_emb[ob]).wait()

    for u in range(_DPW):
        d = wid + u * _NW
        cp_row = pltpu.async_copy(table_hbm.at[d], row_v, s_row)
        cp_pos = pltpu.async_copy(pos_hbm.at[d], pos_v, s_pos)
        for j in range(_NIB):
            pltpu.async_copy(idx_hbm.at[j], idx_bufs[j], s_idx[j])
        cp_pos.wait()
        pltpu.sync_copy(pos_v, pos_out.at[d])
        cp_row.wait()

        for jj in range(_NIB):
            b = jj
            pltpu.make_async_copy(idx_hbm.at[b], idx_bufs[jj], s_idx[jj]).wait()
            if b >= _NOB:
                wait_writes(b - _NOB, jj, d)
            elif u > 0:
                wait_writes(B - _NOB + b, jj, d - _NW)
            compute(b, jj, d)
            pltpu.async_copy(idx_hbm.at[b + _NIB], idx_bufs[jj], s_idx[jj])

        def blk(g, c, d=d):
            for jj in range(_NIB):
                b = g * _NIB + jj
                pltpu.make_async_copy(
                    idx_hbm.at[b], idx_bufs[jj], s_idx[jj]).wait()
                wait_writes(b - _NOB, jj, d)
                compute(b, jj, d)

                @pl.when(b + _NIB < B)
                def _():
                    pltpu.async_copy(
                        idx_hbm.at[b + _NIB], idx_bufs[jj], s_idx[jj])
            return c
        lax.fori_loop(1, B // _NIB, blk, 0)

    d_last = wid + (_DPW - 1) * _NW
    for b in range(B - _NOB, B):
        wait_writes(b, b % _NIB, d_last)


@jax.jit
def _emb_call(idx2d, table_t, pos_t):
    mesh = plsc.VectorSubcoreMesh(core_axis_name="c", subcore_axis_name="s")
    out_type = (
        jax.ShapeDtypeStruct((B, EMB_DIM, T), jnp.float32),
        jax.ShapeDtypeStruct((B, EMB_DIM, T), jnp.float32),
        jax.ShapeDtypeStruct((EMB_DIM, NUM_POS), jnp.float32),
    )
    scratch = [
        pltpu.VMEM((VOC_SIZ,), jnp.float32),
        pltpu.VMEM((T,), jnp.float32),
        [pltpu.VMEM((T,), jnp.int32)] * _NIB,
        [pltpu.VMEM((T,), jnp.float32)] * _NOB,
        [pltpu.VMEM((T,), jnp.float32)] * _NOB,
        pltpu.SemaphoreType.DMA,
        pltpu.SemaphoreType.DMA,
        [pltpu.SemaphoreType.DMA] * _NIB,
        [pltpu.SemaphoreType.DMA] * _NOB,
        [pltpu.SemaphoreType.DMA] * _NOB,
    ]
    fn = functools.partial(
        pl.kernel, mesh=mesh, out_type=out_type, scratch_types=scratch,
        compiler_params=pltpu.CompilerParams(
            needs_layout_passes=False,
            disable_bounds_checks=True,
            disable_semaphore_checks=True,
        ),
    )(_emb_body)
    return fn(idx2d, table_t, pos_t)


def kernel(batInpTokSeq, token_table, pos_table):
    idx2d = batInpTokSeq.astype(jnp.int32)
    emb, tok, pos = _emb_call(idx2d, token_table.T, pos_table.T)
    return (
        emb.transpose(0, 2, 1),
        tok.transpose(0, 2, 1),
        pos.T,
    )
